# baseline (speedup 1.0000x reference)
import functools

import jax
import jax.numpy as jnp
from jax import lax
from jax.experimental import pallas as pl
from jax.experimental.pallas import tpu as pltpu
from jax.experimental.pallas import tpu_sc as plsc

EMBED = 64
LANES = 16
CHUNK = 128


def _sqrt16(a):
    i = lax.bitcast_convert_type(a, jnp.int32)
    y = lax.bitcast_convert_type(jnp.int32(0x5F3759DF) - (i >> 1), jnp.float32)
    for _ in range(3):
        y = y * (1.5 - (0.5 * a * y) * y)
    return a * y


@functools.lru_cache(maxsize=None)
def _build(nw, nc, bpw):
    nchunks = bpw // CHUNK
    ngroups = bpw // LANES
    mesh = plsc.VectorSubcoreMesh(core_axis_name="c", subcore_axis_name="s")

    @functools.partial(
        pl.kernel,
        out_type=jax.ShapeDtypeStruct((nw, bpw), jnp.float32),
        mesh=mesh,
        compiler_params=pltpu.CompilerParams(
            needs_layout_passes=False, use_tc_tiling_on_sc=False
        ),
        scratch_types=[
            pltpu.VMEM((nchunks, CHUNK), jnp.int32),
            pltpu.VMEM((nchunks, CHUNK), jnp.int32),
            pltpu.VMEM((nchunks, CHUNK), jnp.int32),
            pltpu.VMEM((bpw, EMBED), jnp.float32),
            pltpu.VMEM((bpw, EMBED), jnp.float32),
            pltpu.VMEM((bpw, EMBED), jnp.float32),
            pltpu.VMEM((bpw,), jnp.float32),
            pltpu.SemaphoreType.DMA,
        ],
    )
    def trans_e(ent, rel, head, rela, tail, out,
                hidx, ridx, tidx, hrows, rrows, trows, outv, sem):
        wid = lax.axis_index("s") * nc + lax.axis_index("c")

        pltpu.sync_copy(head.at[wid], hidx)
        pltpu.sync_copy(rela.at[wid], ridx)
        pltpu.sync_copy(tail.at[wid], tidx)

        copies = []
        for c in range(nchunks):
            sl = pl.ds(c * CHUNK, CHUNK)
            copies.append(pltpu.async_copy(ent.at[hidx.at[c]], hrows.at[sl], sem))
            copies.append(pltpu.async_copy(rel.at[ridx.at[c]], rrows.at[sl], sem))
            copies.append(pltpu.async_copy(ent.at[tidx.at[c]], trows.at[sl], sem))
        for cp in copies:
            cp.wait()

        def gbody(g, carry):
            rows16 = g * LANES + lax.iota(jnp.int32, LANES)

            def jbody(j, acc):
                for u in range(4):
                    col = jnp.full((LANES,), j * 4 + u, jnp.int32)
                    hv = plsc.load_gather(hrows, [rows16, col])
                    rv = plsc.load_gather(rrows, [rows16, col])
                    tv = plsc.load_gather(trows, [rows16, col])
                    d = (hv + rv) - tv
                    acc = acc + d * d
                return acc

            acc = lax.fori_loop(0, EMBED // 4, jbody, jnp.zeros((LANES,), jnp.float32))
            outv[pl.ds(g * LANES, LANES)] = _sqrt16(acc)
            return carry

        lax.fori_loop(0, ngroups, gbody, 0)
        pltpu.sync_copy(outv, out.at[wid])

    return trans_e


def kernel(entity_embeddings, relation_embeddings, head, relation, tail):
    info = plsc.get_sparse_core_info()
    nw = info.num_cores * info.num_subcores
    batch = head.shape[0]
    bpw = batch // nw
    fn = _build(nw, info.num_cores, bpw)
    nchunks = bpw // CHUNK
    head_r = head.reshape(nw, nchunks, CHUNK)
    rel_r = relation.reshape(nw, nchunks, CHUNK)
    tail_r = tail.reshape(nw, nchunks, CHUNK)
    out = fn(entity_embeddings, relation_embeddings, head_r, rel_r, tail_r)
    return out.reshape(batch)
